# baseline (speedup 1.0000x reference)
import functools

import jax
import jax.numpy as jnp
from jax import lax
from jax.experimental import pallas as pl
from jax.experimental.pallas import tpu as pltpu
from jax.experimental.pallas import tpu_sc as plsc

_B, _N, _G = 4, 5000, 32
_M = _N + _G
_MP = 5120
_CHUNK = _MP // 8
_NV = _CHUNK // 16
_RB = 256
_FG_CAP = 64
_FG_THRESH = 0.5
_BG_HI = 0.5
_BG_LO = 0.0
_CH = 16
_LN2 = 0.6931471805599453
_SEC = (0, 64, 320, 576)
_BLK = 832
_BLKW = _BLK * _CH


def _vlog(x):
    bits = lax.bitcast_convert_type(x, jnp.int32)
    e = jnp.bitwise_and(jnp.right_shift(bits, 23), 255) - 127
    mbits = jnp.bitwise_or(jnp.bitwise_and(bits, 0x007FFFFF), 0x3F800000)
    m = lax.bitcast_convert_type(mbits, jnp.float32)
    big = m > 1.4142135381698608
    m = jnp.where(big, m * 0.5, m)
    e = e + jnp.where(big, 1, 0)
    s = (m - 1.0) / (m + 1.0)
    z = s * s
    p = 2.0 + z * (0.66666666666 + z * (0.4 + z * (0.28571428571
        + z * (0.22222222222 + z * 0.18181818181))))
    return e.astype(jnp.float32) * _LN2 + s * p


def _emit_body(rois_hbm, gt_hbm, bird_hbm,
               oroi_hbm, obox_hbm, olab_hbm, ofrq_hbm,
               rois_v, gt_v, gtb_v, bird_v, maxov_v, assign_v,
               rowsf_v, exct_v, cnt_v, cntall_v, outf_v, och_v, olab_v, sem,
               counts_sp, big_sp):
    c = lax.axis_index("c")
    s = lax.axis_index("s")
    im_local = jnp.right_shift(s, 3)
    im = 2 * c + im_local
    chunk = jnp.bitwise_and(s, 7)
    base = chunk * _CHUNK
    lane = lax.iota(jnp.int32, 16)
    zf = jnp.zeros((16,), jnp.float32)
    zi = jnp.zeros((16,), jnp.int32)

    @pl.when(chunk < 7)
    def _():
        rcp = pltpu.async_copy(
            rois_hbm.at[pl.ds(pl.multiple_of(im * (_N * 4) + base * 4, 8),
                              _CHUNK * 4)],
            rois_v, sem)
        pltpu.sync_copy(gt_hbm.at[pl.ds(im * (_G * 4), _G * 4)], gt_v)
        pltpu.sync_copy(bird_hbm.at[pl.ds(im * _G, _G)], bird_v)
        rcp.wait()

    @pl.when(chunk == 7)
    def _():
        rcp = pltpu.async_copy(
            rois_hbm.at[pl.ds(pl.multiple_of(im * (_N * 4) + base * 4, 8),
                              (_N - 7 * _CHUNK) * 4)],
            rois_v.at[pl.ds(0, (_N - 7 * _CHUNK) * 4)], sem)
        pltpu.sync_copy(gt_hbm.at[pl.ds(im * (_G * 4), _G * 4)], gt_v)
        pltpu.sync_copy(bird_hbm.at[pl.ds(im * _G, _G)], bird_v)
        gtp = (_N - 7 * _CHUNK) * 4
        pltpu.sync_copy(gt_hbm.at[pl.ds(im * (_G * 4), _G * 4)],
                        rois_v.at[pl.ds(gtp, _G * 4)])
        rcp.wait()

    def _bcast(g, carry):
        gs = jnp.full((16,), g, jnp.int32)
        v = []
        for cc in range(4):
            v.append(plsc.load_gather(gt_v, [gs * 4 + cc]))
            gtb_v[cc, g] = v[cc]
        gtb_v[4, g] = (v[2] - v[0]) * (v[3] - v[1])
        return carry
    lax.fori_loop(0, _G, _bcast, 0)

    def _pass1(i, carry):
        n_fg, n_bg, n_ot = carry
        off = i * 16
        gidx = base + off + lane
        valid = gidx < _M
        slotv = (off + lane) * 4
        x1 = plsc.load_gather(rois_v, [slotv])
        y1 = plsc.load_gather(rois_v, [slotv + 1])
        x2 = plsc.load_gather(rois_v, [slotv + 2])
        y2 = plsc.load_gather(rois_v, [slotv + 3])
        area_a = (x2 - x1) * (y2 - y1)
        cur = jnp.full((16,), -1.0, jnp.float32)
        asg = zi
        for g in range(_G):
            gx1 = gtb_v[0, g]
            gy1 = gtb_v[1, g]
            gx2 = gtb_v[2, g]
            gy2 = gtb_v[3, g]
            ga = gtb_v[4, g]
            w = jnp.minimum(x2, gx2) - jnp.maximum(x1, gx1)
            h = jnp.minimum(y2, gy2) - jnp.maximum(y1, gy1)
            inter = jnp.maximum(w, 0.0) * jnp.maximum(h, 0.0)
            den = area_a + ga - inter + 1e-8
            iou = inter / den
            better = iou > cur
            cur = jnp.where(better, iou, cur)
            asg = jnp.where(better, g, asg)
        maxov_v[pl.ds(off, 16)] = cur
        assign_v[pl.ds(off, 16)] = asg
        fg = valid & (cur > _FG_THRESH)
        bg = valid & (cur < _BG_HI) & (cur >= _BG_LO)
        ot = valid & jnp.logical_not(fg | bg)
        n_fg = n_fg + jnp.sum(jnp.where(fg, 1, 0))
        n_bg = n_bg + jnp.sum(jnp.where(bg, 1, 0))
        n_ot = n_ot + jnp.sum(jnp.where(ot, 1, 0))
        return (n_fg, n_bg, n_ot)

    zero = jnp.zeros((), jnp.int32)
    n_fg, n_bg, n_ot = lax.fori_loop(0, _NV, _pass1, (zero, zero, zero))

    cvec = (jnp.where(lane == 0, n_fg, 0) + jnp.where(lane == 1, n_bg, 0)
            + jnp.where(lane == 2, n_ot, 0))
    cnt_v[...] = cvec
    pltpu.sync_copy(cnt_v, counts_sp.at[pl.ds(s * 16, 16)])
    plsc.subcore_barrier()
    pltpu.sync_copy(counts_sp, cntall_v)

    rowsel = jnp.minimum(8 * im_local + lane, 15)
    in_img = lane < 8
    before = lane < chunk

    def _col(ccol):
        v = plsc.load_gather(cntall_v, [rowsel * 16 + ccol])
        vm = jnp.where(in_img, v, 0)
        tot = jnp.sum(vm)
        exc = jnp.sum(jnp.where(before, v, 0))
        evec = plsc.cumsum(vm) - vm
        return tot, exc, evec

    tot_fg, exc_fg, ev_fg = _col(0)
    tot_bg, exc_bg, ev_bg = _col(1)
    tot_ot, exc_ot, ev_ot = _col(2)

    fg_take = jnp.minimum(tot_fg, _FG_CAP)
    bg_take = jnp.minimum(tot_bg, _RB - fg_take)
    ot_take = _RB - fg_take - bg_take
    c2 = jnp.minimum(tot_ot, ot_take)
    off_bg = fg_take
    off_ot = fg_take + bg_take
    off_c3 = fg_take + bg_take + c2
    exc_c3 = (_CHUNK * chunk - jnp.minimum(fg_take, exc_fg)
              - jnp.minimum(bg_take, exc_bg) - jnp.minimum(ot_take, exc_ot))
    ev_c3 = (_CHUNK * lane - jnp.minimum(fg_take, ev_fg)
             - jnp.minimum(bg_take, ev_bg) - jnp.minimum(ot_take, ev_ot))

    exct_v[pl.ds(0, 16)] = ev_fg
    exct_v[pl.ds(16, 16)] = ev_bg
    exct_v[pl.ds(32, 16)] = ev_ot
    exct_v[pl.ds(48, 16)] = ev_c3

    def _pass2(i, carry):
        cf, cb, co, c3 = carry
        off = i * 16
        gidx = base + off + lane
        valid = gidx < _M
        cur = maxov_v[pl.ds(off, 16)]
        asg = assign_v[pl.ds(off, 16)]
        fg = valid & (cur > _FG_THRESH)
        bg = valid & (cur < _BG_HI) & (cur >= _BG_LO)
        ot = valid & jnp.logical_not(fg | bg)

        lr_fg = cf + plsc.cumsum(jnp.where(fg, 1, 0)) - 1
        lr_bg = cb + plsc.cumsum(jnp.where(bg, 1, 0)) - 1
        lr_ot = co + plsc.cumsum(jnp.where(ot, 1, 0)) - 1
        sel_fg = fg & (exc_fg + lr_fg < fg_take)
        sel_bg = bg & (exc_bg + lr_bg < bg_take)
        sel_ot = ot & (exc_ot + lr_ot < ot_take)
        c3m = valid & jnp.logical_not(sel_fg | sel_bg | sel_ot)
        lr_c3 = c3 + plsc.cumsum(jnp.where(c3m, 1, 0)) - 1
        sel_c3 = c3m & (off_c3 + exc_c3 + lr_c3 < _RB)
        sel = sel_fg | sel_bg | sel_ot | sel_c3

        cf = cf + jnp.sum(jnp.where(fg, 1, 0))
        cb = cb + jnp.sum(jnp.where(bg, 1, 0))
        co = co + jnp.sum(jnp.where(ot, 1, 0))
        c3 = c3 + jnp.sum(jnp.where(c3m, 1, 0))

        slotrow = jnp.where(sel_fg, _SEC[0] + lr_fg,
                  jnp.where(sel_bg, _SEC[1] + lr_bg,
                  jnp.where(sel_ot, _SEC[2] + lr_ot, _SEC[3] + lr_c3)))
        saddr = slotrow * _CH

        slotv = (off + lane) * 4
        x1 = plsc.load_gather(rois_v, [slotv])
        y1 = plsc.load_gather(rois_v, [slotv + 1])
        x2 = plsc.load_gather(rois_v, [slotv + 2])
        y2 = plsc.load_gather(rois_v, [slotv + 3])
        ax1 = plsc.load_gather(gt_v, [asg * 4])
        ay1 = plsc.load_gather(gt_v, [asg * 4 + 1])
        ax2 = plsc.load_gather(gt_v, [asg * 4 + 2])
        ay2 = plsc.load_gather(gt_v, [asg * 4 + 3])
        bird = plsc.load_gather(bird_v, [asg])
        lab = jnp.where((cur < _FG_THRESH) | ((1 + bird) == 0), 0, 1)
        labf = lab.astype(jnp.float32)

        ew = x2 - x1 + 1e-8
        eh = y2 - y1 + 1e-8
        ecx = x1 + 0.5 * ew
        ecy = y1 + 0.5 * eh
        gw = ax2 - ax1 + 1e-8
        gh = ay2 - ay1 + 1e-8
        gcx = ax1 + 0.5 * gw
        gcy = ay1 + 0.5 * gh
        dx = (gcx - ecx) / ew
        dy = (gcy - ecy) / eh
        dw = _vlog(gw / ew)
        dh = _vlog(gh / eh)
        freq = (y1 + 0.5 * (y2 - y1)) * (1.0 / 512.0)

        zf16 = jnp.zeros((16,), jnp.float32)
        for col, val in ((0, x1), (1, y1), (2, x2), (3, y2),
                         (4, zf16), (5, zf16), (6, zf16), (7, zf16),
                         (8, dx * labf), (9, dy * labf),
                         (10, dw * labf), (11, dh * labf),
                         (12, labf), (13, freq)):
            plsc.store_scatter(rowsf_v, [saddr + col], val, mask=sel)
        return (cf, cb, co, c3)

    lax.fori_loop(0, _NV, _pass2, (zero, zero, zero, zero))

    pltpu.sync_copy(rowsf_v, big_sp.at[pl.ds(s * _BLKW, _BLKW)])
    plsc.subcore_barrier()

    copies = []
    for k2 in range(2):
        posm = s * 32 + lane + 16 * k2 - im_local * _RB
        isfg = posm < fg_take
        isbg = jnp.logical_not(isfg) & (posm < off_ot)
        isot = jnp.logical_not(isfg | isbg) & (posm < off_c3)
        rcat = posm - jnp.where(isfg, 0,
                      jnp.where(isbg, off_bg,
                      jnp.where(isot, off_ot, off_c3)))
        tbase = jnp.where(isfg, 0,
                jnp.where(isbg, 16,
                jnp.where(isot, 32, 48)))
        secb = jnp.where(isfg, _SEC[0],
               jnp.where(isbg, _SEC[1],
               jnp.where(isot, _SEC[2], _SEC[3])))
        acc = zi
        for w in range(1, 8):
            e_w = plsc.load_gather(exct_v, [tbase + w])
            acc = acc + jnp.where(e_w <= rcat, 1, 0)
        e_star = plsc.load_gather(exct_v, [tbase + acc])
        slot = rcat - e_star
        srcaddr = ((8 * im_local + acc) * _BLK + secb + slot) * _CH
        for k in range(16):
            copies.append(pltpu.async_copy(
                big_sp.at[pl.ds(pl.multiple_of(srcaddr[k], 16), 16)],
                outf_v.at[pl.ds((16 * k2 + k) * 16, 16)], sem))
    for cp in copies:
        cp.wait()

    r0 = c * 512 + s * 32
    for jj in range(8):
        wv = lane + 16 * jj
        och_v[pl.ds(16 * jj, 16)] = plsc.load_gather(
            outf_v, [jnp.right_shift(wv, 2) * _CH + jnp.bitwise_and(wv, 3)])
    rc = pltpu.async_copy(och_v.at[pl.ds(0, 128)],
                          oroi_hbm.at[pl.ds(r0 * 4, 128)], sem)
    for jj in range(16):
        wv = lane + 16 * jj
        och_v[pl.ds(128 + 16 * jj, 16)] = plsc.load_gather(
            outf_v, [jnp.right_shift(wv, 3) * _CH + 4 + jnp.bitwise_and(wv, 7)])
    bc = pltpu.async_copy(och_v.at[pl.ds(128, 256)],
                          obox_hbm.at[pl.ds(r0 * 8, 256)], sem)
    for jj in range(2):
        wv = lane + 16 * jj
        olab_v[pl.ds(16 * jj, 16)] = plsc.load_gather(
            outf_v, [wv * _CH + 12]).astype(jnp.int32)
        och_v[pl.ds(384 + 16 * jj, 16)] = plsc.load_gather(
            outf_v, [wv * _CH + 13])
    lc = pltpu.async_copy(olab_v, olab_hbm.at[pl.ds(r0, 32)], sem)
    fc = pltpu.async_copy(och_v.at[pl.ds(384, 32)],
                          ofrq_hbm.at[pl.ds(r0, 32)], sem)
    rc.wait(); bc.wait(); lc.wait(); fc.wait()


@functools.cache
def _sc_call():
    return pl.kernel(
        _emit_body,
        out_type=(jax.ShapeDtypeStruct((_B * _RB * 4,), jnp.float32),
                  jax.ShapeDtypeStruct((_B * _RB * 8,), jnp.float32),
                  jax.ShapeDtypeStruct((_B * _RB,), jnp.int32),
                  jax.ShapeDtypeStruct((_B * _RB,), jnp.float32)),
        mesh=plsc.VectorSubcoreMesh(core_axis_name="c", subcore_axis_name="s"),
        compiler_params=pltpu.CompilerParams(needs_layout_passes=False),
        scratch_types=[
            pltpu.VMEM((4 * _CHUNK,), jnp.float32),
            pltpu.VMEM((4 * _G,), jnp.float32),
            pltpu.VMEM((5, _G, 16), jnp.float32),
            pltpu.VMEM((_G,), jnp.int32),
            pltpu.VMEM((_CHUNK,), jnp.float32),
            pltpu.VMEM((_CHUNK,), jnp.int32),
            pltpu.VMEM((_BLKW,), jnp.float32),
            pltpu.VMEM((64,), jnp.int32),
            pltpu.VMEM((16,), jnp.int32),
            pltpu.VMEM((256,), jnp.int32),
            pltpu.VMEM((512,), jnp.float32),
            pltpu.VMEM((448,), jnp.float32),
            pltpu.VMEM((32,), jnp.int32),
            pltpu.SemaphoreType.DMA,
            pltpu.VMEM_SHARED((256,), jnp.int32),
            pltpu.VMEM_SHARED((16 * _BLKW,), jnp.float32),
        ],
    )


def kernel(rois, gt_bbox, bird_ids, lengths):
    del lengths
    oroi, obox, olab, ofrq = _sc_call()(
        rois.astype(jnp.float32).reshape(-1),
        gt_bbox.astype(jnp.float32).reshape(-1),
        bird_ids.astype(jnp.int32).reshape(-1))
    return (oroi.reshape(_B, _RB, 4), obox.reshape(_B, _RB, 8),
            olab.reshape(_B, _RB), ofrq.reshape(_B, _RB, 1))


# speedup vs baseline: 1.3520x; 1.3520x over previous
import functools

import jax
import jax.numpy as jnp
from jax import lax
from jax.experimental import pallas as pl
from jax.experimental.pallas import tpu as pltpu
from jax.experimental.pallas import tpu_sc as plsc

_B, _N, _G = 4, 5000, 32
_M = _N + _G
_MP = 5120
_CHUNK = _MP // 8
_NV = _CHUNK // 16
_RB = 256
_FG_CAP = 64
_FG_THRESH = 0.5
_BG_HI = 0.5
_BG_LO = 0.0
_CH = 16
_LN2 = 0.6931471805599453
_SEC = (0, 64, 320, 576)
_BLK = 832
_BLKW = _BLK * _CH


def _vlog(x):
    bits = lax.bitcast_convert_type(x, jnp.int32)
    e = jnp.bitwise_and(jnp.right_shift(bits, 23), 255) - 127
    mbits = jnp.bitwise_or(jnp.bitwise_and(bits, 0x007FFFFF), 0x3F800000)
    m = lax.bitcast_convert_type(mbits, jnp.float32)
    big = m > 1.4142135381698608
    m = jnp.where(big, m * 0.5, m)
    e = e + jnp.where(big, 1, 0)
    s = (m - 1.0) / (m + 1.0)
    z = s * s
    p = 2.0 + z * (0.66666666666 + z * (0.4 + z * (0.28571428571
        + z * (0.22222222222 + z * 0.18181818181))))
    return e.astype(jnp.float32) * _LN2 + s * p


def _emit_body(rois_hbm, gt_hbm, bird_hbm, out_hbm,
               rois_v, gt_v, gtb_v, bird_v, maxov_v, assign_v,
               rowsf_v, exct_v, cnt_v, cntall_v, outf_v, sem,
               counts_sp, big_sp):
    c = lax.axis_index("c")
    s = lax.axis_index("s")
    im_local = jnp.right_shift(s, 3)
    im = 2 * c + im_local
    chunk = jnp.bitwise_and(s, 7)
    base = chunk * _CHUNK
    lane = lax.iota(jnp.int32, 16)
    zf = jnp.zeros((16,), jnp.float32)
    zi = jnp.zeros((16,), jnp.int32)

    rcps = [pltpu.async_copy(
        rois_hbm.at[pl.ds((4 * im + cc) * _MP + base, _CHUNK)],
        rois_v.at[cc], sem) for cc in range(4)]
    for cc in range(4):
        pltpu.sync_copy(gt_hbm.at[pl.ds((4 * im + cc) * _G, _G)],
                        gt_v.at[pl.ds(cc * _G, _G)])
    pltpu.sync_copy(bird_hbm.at[pl.ds(im * _G, _G)], bird_v)

    def _bcast(g, carry):
        gs = jnp.full((16,), g, jnp.int32)
        v = []
        for cc in range(4):
            v.append(plsc.load_gather(gt_v, [gs + cc * _G]))
            gtb_v[cc, g] = v[cc]
        gtb_v[4, g] = (v[2] - v[0]) * (v[3] - v[1])
        return carry
    lax.fori_loop(0, _G, _bcast, 0)
    for rcp in rcps:
        rcp.wait()

    def _pass1(i, carry):
        n_fg, n_bg, n_ot = carry
        off = i * 16
        gidx = base + off + lane
        valid = gidx < _M
        x1 = rois_v[0, pl.ds(off, 16)]
        y1 = rois_v[1, pl.ds(off, 16)]
        x2 = rois_v[2, pl.ds(off, 16)]
        y2 = rois_v[3, pl.ds(off, 16)]
        area_a = (x2 - x1) * (y2 - y1)
        cur = jnp.full((16,), -1.0, jnp.float32)
        asg = zi
        for g in range(_G):
            gx1 = gtb_v[0, g]
            gy1 = gtb_v[1, g]
            gx2 = gtb_v[2, g]
            gy2 = gtb_v[3, g]
            ga = gtb_v[4, g]
            w = jnp.minimum(x2, gx2) - jnp.maximum(x1, gx1)
            h = jnp.minimum(y2, gy2) - jnp.maximum(y1, gy1)
            inter = jnp.maximum(w, 0.0) * jnp.maximum(h, 0.0)
            den = area_a + ga - inter + 1e-8
            iou = inter / den
            better = iou > cur
            cur = jnp.where(better, iou, cur)
            asg = jnp.where(better, g, asg)
        maxov_v[pl.ds(off, 16)] = cur
        assign_v[pl.ds(off, 16)] = asg
        fg = valid & (cur > _FG_THRESH)
        bg = valid & (cur < _BG_HI) & (cur >= _BG_LO)
        ot = valid & jnp.logical_not(fg | bg)
        n_fg = n_fg + jnp.sum(jnp.where(fg, 1, 0))
        n_bg = n_bg + jnp.sum(jnp.where(bg, 1, 0))
        n_ot = n_ot + jnp.sum(jnp.where(ot, 1, 0))
        return (n_fg, n_bg, n_ot)

    zero = jnp.zeros((), jnp.int32)
    n_fg, n_bg, n_ot = lax.fori_loop(0, _NV, _pass1, (zero, zero, zero))

    cvec = (jnp.where(lane == 0, n_fg, 0) + jnp.where(lane == 1, n_bg, 0)
            + jnp.where(lane == 2, n_ot, 0))
    cnt_v[...] = cvec
    pltpu.sync_copy(cnt_v, counts_sp.at[pl.ds(s * 16, 16)])
    plsc.subcore_barrier()
    pltpu.sync_copy(counts_sp, cntall_v)

    rowsel = jnp.minimum(8 * im_local + lane, 15)
    in_img = lane < 8
    before = lane < chunk

    def _col(ccol):
        v = plsc.load_gather(cntall_v, [rowsel * 16 + ccol])
        vm = jnp.where(in_img, v, 0)
        tot = jnp.sum(vm)
        exc = jnp.sum(jnp.where(before, v, 0))
        evec = plsc.cumsum(vm) - vm
        return tot, exc, evec

    tot_fg, exc_fg, ev_fg = _col(0)
    tot_bg, exc_bg, ev_bg = _col(1)
    tot_ot, exc_ot, ev_ot = _col(2)

    fg_take = jnp.minimum(tot_fg, _FG_CAP)
    bg_take = jnp.minimum(tot_bg, _RB - fg_take)
    ot_take = _RB - fg_take - bg_take
    c2 = jnp.minimum(tot_ot, ot_take)
    off_bg = fg_take
    off_ot = fg_take + bg_take
    off_c3 = fg_take + bg_take + c2
    exc_c3 = (_CHUNK * chunk - jnp.minimum(fg_take, exc_fg)
              - jnp.minimum(bg_take, exc_bg) - jnp.minimum(ot_take, exc_ot))
    ev_c3 = (_CHUNK * lane - jnp.minimum(fg_take, ev_fg)
             - jnp.minimum(bg_take, ev_bg) - jnp.minimum(ot_take, ev_ot))

    exct_v[pl.ds(0, 16)] = ev_fg
    exct_v[pl.ds(16, 16)] = ev_bg
    exct_v[pl.ds(32, 16)] = ev_ot
    exct_v[pl.ds(48, 16)] = ev_c3

    def _pass2(i, carry):
        cf, cb, co, c3 = carry
        off = i * 16
        gidx = base + off + lane
        valid = gidx < _M
        cur = maxov_v[pl.ds(off, 16)]
        asg = assign_v[pl.ds(off, 16)]
        fg = valid & (cur > _FG_THRESH)
        bg = valid & (cur < _BG_HI) & (cur >= _BG_LO)
        ot = valid & jnp.logical_not(fg | bg)

        lr_fg = cf + plsc.cumsum(jnp.where(fg, 1, 0)) - 1
        lr_bg = cb + plsc.cumsum(jnp.where(bg, 1, 0)) - 1
        lr_ot = co + plsc.cumsum(jnp.where(ot, 1, 0)) - 1
        sel_fg = fg & (exc_fg + lr_fg < fg_take)
        sel_bg = bg & (exc_bg + lr_bg < bg_take)
        sel_ot = ot & (exc_ot + lr_ot < ot_take)
        c3m = valid & jnp.logical_not(sel_fg | sel_bg | sel_ot)
        lr_c3 = c3 + plsc.cumsum(jnp.where(c3m, 1, 0)) - 1
        sel_c3 = c3m & (off_c3 + exc_c3 + lr_c3 < _RB)
        sel = sel_fg | sel_bg | sel_ot | sel_c3

        cf = cf + jnp.sum(jnp.where(fg, 1, 0))
        cb = cb + jnp.sum(jnp.where(bg, 1, 0))
        co = co + jnp.sum(jnp.where(ot, 1, 0))
        c3 = c3 + jnp.sum(jnp.where(c3m, 1, 0))

        slotrow = jnp.where(sel_fg, _SEC[0] + lr_fg,
                  jnp.where(sel_bg, _SEC[1] + lr_bg,
                  jnp.where(sel_ot, _SEC[2] + lr_ot, _SEC[3] + lr_c3)))
        saddr = slotrow * _CH

        x1 = rois_v[0, pl.ds(off, 16)]
        y1 = rois_v[1, pl.ds(off, 16)]
        x2 = rois_v[2, pl.ds(off, 16)]
        y2 = rois_v[3, pl.ds(off, 16)]
        ax1 = plsc.load_gather(gt_v, [asg])
        ay1 = plsc.load_gather(gt_v, [asg + _G])
        ax2 = plsc.load_gather(gt_v, [asg + 2 * _G])
        ay2 = plsc.load_gather(gt_v, [asg + 3 * _G])
        bird = plsc.load_gather(bird_v, [asg])
        lab = jnp.where((cur < _FG_THRESH) | ((1 + bird) == 0), 0, 1)
        labf = lab.astype(jnp.float32)

        ew = x2 - x1 + 1e-8
        eh = y2 - y1 + 1e-8
        ecx = x1 + 0.5 * ew
        ecy = y1 + 0.5 * eh
        gw = ax2 - ax1 + 1e-8
        gh = ay2 - ay1 + 1e-8
        gcx = ax1 + 0.5 * gw
        gcy = ay1 + 0.5 * gh
        dx = (gcx - ecx) / ew
        dy = (gcy - ecy) / eh
        dw = _vlog(gw / ew)
        dh = _vlog(gh / eh)
        freq = (y1 + 0.5 * (y2 - y1)) * (1.0 / 512.0)

        zf16 = jnp.zeros((16,), jnp.float32)
        for col, val in ((0, x1), (1, y1), (2, x2), (3, y2),
                         (4, zf16), (5, zf16), (6, zf16), (7, zf16),
                         (8, dx * labf), (9, dy * labf),
                         (10, dw * labf), (11, dh * labf),
                         (12, labf), (13, freq)):
            plsc.store_scatter(rowsf_v, [saddr + col], val, mask=sel)
        return (cf, cb, co, c3)

    lax.fori_loop(0, _NV, _pass2, (zero, zero, zero, zero))

    pltpu.sync_copy(rowsf_v, big_sp.at[pl.ds(s * _BLKW, _BLKW)])
    plsc.subcore_barrier()

    copies = []
    for k2 in range(2):
        posm = s * 32 + lane + 16 * k2 - im_local * _RB
        isfg = posm < fg_take
        isbg = jnp.logical_not(isfg) & (posm < off_ot)
        isot = jnp.logical_not(isfg | isbg) & (posm < off_c3)
        rcat = posm - jnp.where(isfg, 0,
                      jnp.where(isbg, off_bg,
                      jnp.where(isot, off_ot, off_c3)))
        tbase = jnp.where(isfg, 0,
                jnp.where(isbg, 16,
                jnp.where(isot, 32, 48)))
        secb = jnp.where(isfg, _SEC[0],
               jnp.where(isbg, _SEC[1],
               jnp.where(isot, _SEC[2], _SEC[3])))
        acc = zi
        for w in range(1, 8):
            e_w = plsc.load_gather(exct_v, [tbase + w])
            acc = acc + jnp.where(e_w <= rcat, 1, 0)
        e_star = plsc.load_gather(exct_v, [tbase + acc])
        slot = rcat - e_star
        srcaddr = ((8 * im_local + acc) * _BLK + secb + slot) * _CH
        for k in range(16):
            copies.append(pltpu.async_copy(
                big_sp.at[pl.ds(pl.multiple_of(srcaddr[k], 16), 16)],
                outf_v.at[pl.ds((16 * k2 + k) * 16, 16)], sem))
    for cp in copies:
        cp.wait()

    pltpu.sync_copy(outf_v, out_hbm.at[pl.ds((c * 512 + s * 32) * _CH, 512)])


@functools.cache
def _sc_call():
    return pl.kernel(
        _emit_body,
        out_type=jax.ShapeDtypeStruct((_B * _RB * _CH,), jnp.float32),
        mesh=plsc.VectorSubcoreMesh(core_axis_name="c", subcore_axis_name="s"),
        compiler_params=pltpu.CompilerParams(needs_layout_passes=False),
        scratch_types=[
            pltpu.VMEM((4, _CHUNK), jnp.float32),
            pltpu.VMEM((4 * _G,), jnp.float32),
            pltpu.VMEM((5, _G, 16), jnp.float32),
            pltpu.VMEM((_G,), jnp.int32),
            pltpu.VMEM((_CHUNK,), jnp.float32),
            pltpu.VMEM((_CHUNK,), jnp.int32),
            pltpu.VMEM((_BLKW,), jnp.float32),
            pltpu.VMEM((64,), jnp.int32),
            pltpu.VMEM((16,), jnp.int32),
            pltpu.VMEM((256,), jnp.int32),
            pltpu.VMEM((512,), jnp.float32),
            pltpu.SemaphoreType.DMA,
            pltpu.VMEM_SHARED((256,), jnp.int32),
            pltpu.VMEM_SHARED((16 * _BLKW,), jnp.float32),
        ],
    )


def kernel(rois, gt_bbox, bird_ids, lengths):
    del lengths
    gt = gt_bbox.reshape(_B, _G, 4).astype(jnp.float32)
    allr = jnp.concatenate([rois.astype(jnp.float32), gt], axis=1)
    allr = jnp.pad(allr, ((0, 0), (0, _MP - _M), (0, 0)))
    rois_soa = allr.transpose(0, 2, 1).reshape(-1)
    gt_soa = gt.transpose(0, 2, 1).reshape(-1)
    bird = bird_ids.reshape(-1).astype(jnp.int32)
    packed = _sc_call()(rois_soa, gt_soa, bird).reshape(_B, _RB, _CH)
    out_rois = packed[:, :, 0:4]
    bbox_targets = packed[:, :, 4:12]
    out_labels = packed[:, :, 12].astype(jnp.int32)
    freq_targets = packed[:, :, 13:14]
    return (out_rois, bbox_targets, out_labels, freq_targets)


# speedup vs baseline: 1.4206x; 1.0508x over previous
import functools

import jax
import jax.numpy as jnp
from jax import lax
from jax.experimental import pallas as pl
from jax.experimental.pallas import tpu as pltpu
from jax.experimental.pallas import tpu_sc as plsc

_B, _N, _G = 4, 5000, 32
_M = _N + _G
_MP = 5120
_CHUNK = _MP // 8
_NV = _CHUNK // 16
_RB = 256
_FG_CAP = 64
_FG_THRESH = 0.5
_BG_HI = 0.5
_BG_LO = 0.0
_CH = 16
_LN2 = 0.6931471805599453
_SEC = (0, 64, 320, 576)
_BLK = 832
_BLKW = _BLK * _CH


def _vlog(x):
    bits = lax.bitcast_convert_type(x, jnp.int32)
    e = jnp.bitwise_and(jnp.right_shift(bits, 23), 255) - 127
    mbits = jnp.bitwise_or(jnp.bitwise_and(bits, 0x007FFFFF), 0x3F800000)
    m = lax.bitcast_convert_type(mbits, jnp.float32)
    big = m > 1.4142135381698608
    m = jnp.where(big, m * 0.5, m)
    e = e + jnp.where(big, 1, 0)
    s = (m - 1.0) / (m + 1.0)
    z = s * s
    p = 2.0 + z * (0.66666666666 + z * (0.4 + z * (0.28571428571
        + z * (0.22222222222 + z * 0.18181818181))))
    return e.astype(jnp.float32) * _LN2 + s * p


def _emit_body(rois_hbm, gt_hbm, bird_hbm, out_hbm,
               rois_v, gt_v, gtb_v, bird_v, maxov_v, assign_v,
               rowsf_v, exct_v, cnt_v, cntall_v, outf_v, sem,
               counts_sp, big_sp):
    c = lax.axis_index("c")
    s = lax.axis_index("s")
    im_local = jnp.right_shift(s, 3)
    im = 2 * c + im_local
    chunk = jnp.bitwise_and(s, 7)
    base = chunk * _CHUNK
    lane = lax.iota(jnp.int32, 16)
    zf = jnp.zeros((16,), jnp.float32)
    zi = jnp.zeros((16,), jnp.int32)

    rcps = [pltpu.async_copy(
        rois_hbm.at[pl.ds((4 * im + cc) * _MP + base, _CHUNK)],
        rois_v.at[cc], sem) for cc in range(4)]
    for cc in range(4):
        pltpu.sync_copy(gt_hbm.at[pl.ds((4 * im + cc) * _G, _G)],
                        gt_v.at[pl.ds(cc * _G, _G)])
    pltpu.sync_copy(bird_hbm.at[pl.ds(im * _G, _G)], bird_v)

    def _bcast(g, carry):
        gs = jnp.full((16,), g, jnp.int32)
        v = []
        for cc in range(4):
            v.append(plsc.load_gather(gt_v, [gs + cc * _G]))
            gtb_v[cc, g] = v[cc]
        gtb_v[4, g] = (v[2] - v[0]) * (v[3] - v[1])
        return carry
    lax.fori_loop(0, _G, _bcast, 0)
    for rcp in rcps:
        rcp.wait()

    def _pass1(i, carry):
        n_fg, n_bg, n_ot = carry
        off = i * 16
        gidx = base + off + lane
        valid = gidx < _M
        x1 = rois_v[0, pl.ds(off, 16)]
        y1 = rois_v[1, pl.ds(off, 16)]
        x2 = rois_v[2, pl.ds(off, 16)]
        y2 = rois_v[3, pl.ds(off, 16)]
        area_a = (x2 - x1) * (y2 - y1)
        cur = jnp.full((16,), -1.0, jnp.float32)
        asg = zi
        for g in range(_G):
            gx1 = gtb_v[0, g]
            gy1 = gtb_v[1, g]
            gx2 = gtb_v[2, g]
            gy2 = gtb_v[3, g]
            ga = gtb_v[4, g]
            w = jnp.minimum(x2, gx2) - jnp.maximum(x1, gx1)
            h = jnp.minimum(y2, gy2) - jnp.maximum(y1, gy1)
            inter = jnp.maximum(w, 0.0) * jnp.maximum(h, 0.0)
            den = area_a + ga - inter + 1e-8
            iou = inter / den
            better = iou > cur
            cur = jnp.where(better, iou, cur)
            asg = jnp.where(better, g, asg)
        maxov_v[pl.ds(off, 16)] = cur
        assign_v[pl.ds(off, 16)] = asg
        fg = valid & (cur > _FG_THRESH)
        bg = valid & (cur < _BG_HI) & (cur >= _BG_LO)
        n_fg = n_fg + jnp.where(fg, 1, 0)
        n_bg = n_bg + jnp.where(bg, 1, 0)
        return (n_fg, n_bg, n_ot)

    zero = jnp.zeros((), jnp.int32)
    af, ab, _ = lax.fori_loop(0, _NV, _pass1, (zi, zi, zero))
    n_fg = jnp.sum(af)
    n_bg = jnp.sum(ab)
    nvalid = jnp.where(chunk == 7, _M - 7 * _CHUNK, _CHUNK)
    n_ot = nvalid - n_fg - n_bg

    cvec = (jnp.where(lane == 0, n_fg, 0) + jnp.where(lane == 1, n_bg, 0)
            + jnp.where(lane == 2, n_ot, 0))
    cnt_v[...] = cvec
    pltpu.sync_copy(cnt_v, counts_sp.at[pl.ds(s * 16, 16)])
    plsc.subcore_barrier()
    pltpu.sync_copy(counts_sp, cntall_v)

    rowsel = jnp.minimum(8 * im_local + lane, 15)
    in_img = lane < 8
    before = lane < chunk

    def _col(ccol):
        v = plsc.load_gather(cntall_v, [rowsel * 16 + ccol])
        vm = jnp.where(in_img, v, 0)
        tot = jnp.sum(vm)
        exc = jnp.sum(jnp.where(before, v, 0))
        evec = plsc.cumsum(vm) - vm
        return tot, exc, evec

    tot_fg, exc_fg, ev_fg = _col(0)
    tot_bg, exc_bg, ev_bg = _col(1)
    tot_ot, exc_ot, ev_ot = _col(2)

    fg_take = jnp.minimum(tot_fg, _FG_CAP)
    bg_take = jnp.minimum(tot_bg, _RB - fg_take)
    ot_take = _RB - fg_take - bg_take
    c2 = jnp.minimum(tot_ot, ot_take)
    off_bg = fg_take
    off_ot = fg_take + bg_take
    off_c3 = fg_take + bg_take + c2
    exc_c3 = (_CHUNK * chunk - jnp.minimum(fg_take, exc_fg)
              - jnp.minimum(bg_take, exc_bg) - jnp.minimum(ot_take, exc_ot))
    ev_c3 = (_CHUNK * lane - jnp.minimum(fg_take, ev_fg)
             - jnp.minimum(bg_take, ev_bg) - jnp.minimum(ot_take, ev_ot))

    exct_v[pl.ds(0, 16)] = ev_fg
    exct_v[pl.ds(16, 16)] = ev_bg
    exct_v[pl.ds(32, 16)] = ev_ot
    exct_v[pl.ds(48, 16)] = ev_c3

    def _pass2(i, carry):
        cf, cb, co, c3 = carry
        off = i * 16
        gidx = base + off + lane
        valid = gidx < _M
        cur = maxov_v[pl.ds(off, 16)]
        asg = assign_v[pl.ds(off, 16)]
        fg = valid & (cur > _FG_THRESH)
        bg = valid & (cur < _BG_HI) & (cur >= _BG_LO)
        ot = valid & jnp.logical_not(fg | bg)

        cs_fg = plsc.cumsum(jnp.where(fg, 1, 0))
        cs_bg = plsc.cumsum(jnp.where(bg, 1, 0))
        cs_ot = plsc.cumsum(jnp.where(ot, 1, 0))
        lr_fg = cf + cs_fg - 1
        lr_bg = cb + cs_bg - 1
        lr_ot = co + cs_ot - 1
        sel_fg = fg & (exc_fg + lr_fg < fg_take)
        sel_bg = bg & (exc_bg + lr_bg < bg_take)
        sel_ot = ot & (exc_ot + lr_ot < ot_take)
        c3m = valid & jnp.logical_not(sel_fg | sel_bg | sel_ot)
        cs_c3 = plsc.cumsum(jnp.where(c3m, 1, 0))
        lr_c3 = c3 + cs_c3 - 1
        sel_c3 = c3m & (off_c3 + exc_c3 + lr_c3 < _RB)
        sel = sel_fg | sel_bg | sel_ot | sel_c3

        slotrow = jnp.where(sel_fg, _SEC[0] + lr_fg,
                  jnp.where(sel_bg, _SEC[1] + lr_bg,
                  jnp.where(sel_ot, _SEC[2] + lr_ot, _SEC[3] + lr_c3)))
        saddr = slotrow * _CH

        x1 = rois_v[0, pl.ds(off, 16)]
        y1 = rois_v[1, pl.ds(off, 16)]
        x2 = rois_v[2, pl.ds(off, 16)]
        y2 = rois_v[3, pl.ds(off, 16)]
        ax1 = plsc.load_gather(gt_v, [asg])
        ay1 = plsc.load_gather(gt_v, [asg + _G])
        ax2 = plsc.load_gather(gt_v, [asg + 2 * _G])
        ay2 = plsc.load_gather(gt_v, [asg + 3 * _G])
        bird = plsc.load_gather(bird_v, [asg])
        lab = jnp.where((cur < _FG_THRESH) | ((1 + bird) == 0), 0, 1)
        labf = lab.astype(jnp.float32)

        ew = x2 - x1 + 1e-8
        eh = y2 - y1 + 1e-8
        ecx = x1 + 0.5 * ew
        ecy = y1 + 0.5 * eh
        gw = ax2 - ax1 + 1e-8
        gh = ay2 - ay1 + 1e-8
        gcx = ax1 + 0.5 * gw
        gcy = ay1 + 0.5 * gh
        dx = (gcx - ecx) / ew
        dy = (gcy - ecy) / eh
        dw = _vlog(gw / ew)
        dh = _vlog(gh / eh)
        freq = (y1 + 0.5 * (y2 - y1)) * (1.0 / 512.0)

        zf16 = jnp.zeros((16,), jnp.float32)
        for col, val in ((0, x1), (1, y1), (2, x2), (3, y2),
                         (4, zf16), (5, zf16), (6, zf16), (7, zf16),
                         (8, dx * labf), (9, dy * labf),
                         (10, dw * labf), (11, dh * labf),
                         (12, labf), (13, freq)):
            plsc.store_scatter(rowsf_v, [saddr + col], val, mask=sel)
        return (cf + cs_fg[15], cb + cs_bg[15], co + cs_ot[15], c3 + cs_c3[15])

    lax.fori_loop(0, _NV, _pass2, (zero, zero, zero, zero))

    pltpu.sync_copy(rowsf_v, big_sp.at[pl.ds(s * _BLKW, _BLKW)])
    plsc.subcore_barrier()

    copies = []
    for k2 in range(2):
        posm = s * 32 + lane + 16 * k2 - im_local * _RB
        isfg = posm < fg_take
        isbg = jnp.logical_not(isfg) & (posm < off_ot)
        isot = jnp.logical_not(isfg | isbg) & (posm < off_c3)
        rcat = posm - jnp.where(isfg, 0,
                      jnp.where(isbg, off_bg,
                      jnp.where(isot, off_ot, off_c3)))
        tbase = jnp.where(isfg, 0,
                jnp.where(isbg, 16,
                jnp.where(isot, 32, 48)))
        secb = jnp.where(isfg, _SEC[0],
               jnp.where(isbg, _SEC[1],
               jnp.where(isot, _SEC[2], _SEC[3])))
        acc = zi
        for w in range(1, 8):
            e_w = plsc.load_gather(exct_v, [tbase + w])
            acc = acc + jnp.where(e_w <= rcat, 1, 0)
        e_star = plsc.load_gather(exct_v, [tbase + acc])
        slot = rcat - e_star
        srcaddr = ((8 * im_local + acc) * _BLK + secb + slot) * _CH
        for k in range(16):
            copies.append(pltpu.async_copy(
                big_sp.at[pl.ds(pl.multiple_of(srcaddr[k], 16), 16)],
                outf_v.at[pl.ds((16 * k2 + k) * 16, 16)], sem))
    for cp in copies:
        cp.wait()

    pltpu.sync_copy(outf_v, out_hbm.at[pl.ds((c * 512 + s * 32) * _CH, 512)])


@functools.cache
def _sc_call():
    return pl.kernel(
        _emit_body,
        out_type=jax.ShapeDtypeStruct((_B * _RB * _CH,), jnp.float32),
        mesh=plsc.VectorSubcoreMesh(core_axis_name="c", subcore_axis_name="s"),
        compiler_params=pltpu.CompilerParams(needs_layout_passes=False),
        scratch_types=[
            pltpu.VMEM((4, _CHUNK), jnp.float32),
            pltpu.VMEM((4 * _G,), jnp.float32),
            pltpu.VMEM((5, _G, 16), jnp.float32),
            pltpu.VMEM((_G,), jnp.int32),
            pltpu.VMEM((_CHUNK,), jnp.float32),
            pltpu.VMEM((_CHUNK,), jnp.int32),
            pltpu.VMEM((_BLKW,), jnp.float32),
            pltpu.VMEM((64,), jnp.int32),
            pltpu.VMEM((16,), jnp.int32),
            pltpu.VMEM((256,), jnp.int32),
            pltpu.VMEM((512,), jnp.float32),
            pltpu.SemaphoreType.DMA,
            pltpu.VMEM_SHARED((256,), jnp.int32),
            pltpu.VMEM_SHARED((16 * _BLKW,), jnp.float32),
        ],
    )


def kernel(rois, gt_bbox, bird_ids, lengths):
    del lengths
    gt = gt_bbox.reshape(_B, _G, 4).astype(jnp.float32)
    allr = jnp.concatenate([rois.astype(jnp.float32), gt], axis=1)
    allr = jnp.pad(allr, ((0, 0), (0, _MP - _M), (0, 0)))
    rois_soa = allr.transpose(0, 2, 1).reshape(-1)
    gt_soa = gt.transpose(0, 2, 1).reshape(-1)
    bird = bird_ids.reshape(-1).astype(jnp.int32)
    packed = _sc_call()(rois_soa, gt_soa, bird).reshape(_B, _RB, _CH)
    out_rois = packed[:, :, 0:4]
    bbox_targets = packed[:, :, 4:12]
    out_labels = packed[:, :, 12].astype(jnp.int32)
    freq_targets = packed[:, :, 13:14]
    return (out_rois, bbox_targets, out_labels, freq_targets)


# speedup vs baseline: 1.4229x; 1.0016x over previous
import functools

import jax
import jax.numpy as jnp
from jax import lax
from jax.experimental import pallas as pl
from jax.experimental.pallas import tpu as pltpu
from jax.experimental.pallas import tpu_sc as plsc

_B, _N, _G = 4, 5000, 32
_M = _N + _G
_MP = 5120
_CHUNK = _MP // 8
_NV = _CHUNK // 16
_RB = 256
_FG_CAP = 64
_FG_THRESH = 0.5
_BG_HI = 0.5
_BG_LO = 0.0
_CH = 16
_LN2 = 0.6931471805599453
_SEC = (0, 64, 320, 576)
_BLK = 832
_BLKW = _BLK * _CH


def _vlog(x):
    bits = lax.bitcast_convert_type(x, jnp.int32)
    e = jnp.bitwise_and(jnp.right_shift(bits, 23), 255) - 127
    mbits = jnp.bitwise_or(jnp.bitwise_and(bits, 0x007FFFFF), 0x3F800000)
    m = lax.bitcast_convert_type(mbits, jnp.float32)
    big = m > 1.4142135381698608
    m = jnp.where(big, m * 0.5, m)
    e = e + jnp.where(big, 1, 0)
    s = (m - 1.0) / (m + 1.0)
    z = s * s
    p = 2.0 + z * (0.66666666666 + z * (0.4 + z * (0.28571428571
        + z * (0.22222222222 + z * 0.18181818181))))
    return e.astype(jnp.float32) * _LN2 + s * p


def _emit_body(rois_hbm, gt_hbm, bird_hbm, out_hbm,
               rois_v, gt_v, gtb_v, bird_v, maxov_v, assign_v,
               rowsf_v, exct_v, cnt_v, cntall_v, outf_v, sem,
               counts_sp, big_sp):
    c = lax.axis_index("c")
    s = lax.axis_index("s")
    im_local = jnp.right_shift(s, 3)
    im = 2 * c + im_local
    chunk = jnp.bitwise_and(s, 7)
    base = chunk * _CHUNK
    lane = lax.iota(jnp.int32, 16)
    zf = jnp.zeros((16,), jnp.float32)
    zi = jnp.zeros((16,), jnp.int32)

    rcps = [pltpu.async_copy(
        rois_hbm.at[pl.ds((4 * im + cc) * _MP + base, _CHUNK)],
        rois_v.at[cc], sem) for cc in range(4)]
    for cc in range(4):
        pltpu.sync_copy(gt_hbm.at[pl.ds((4 * im + cc) * _G, _G)],
                        gt_v.at[pl.ds(cc * _G, _G)])
    pltpu.sync_copy(bird_hbm.at[pl.ds(im * _G, _G)], bird_v)

    def _bcast(g, carry):
        gs = jnp.full((16,), g, jnp.int32)
        v = []
        for cc in range(4):
            v.append(plsc.load_gather(gt_v, [gs + cc * _G]))
            gtb_v[cc, g] = v[cc]
        gtb_v[4, g] = (v[2] - v[0]) * (v[3] - v[1])
        return carry
    lax.fori_loop(0, _G, _bcast, 0)
    for rcp in rcps:
        rcp.wait()

    def _pass1(i, carry):
        n_fg, n_bg, n_ot = carry
        offs = [i * 32, i * 32 + 16]
        x1 = [rois_v[0, pl.ds(o, 16)] for o in offs]
        y1 = [rois_v[1, pl.ds(o, 16)] for o in offs]
        x2 = [rois_v[2, pl.ds(o, 16)] for o in offs]
        y2 = [rois_v[3, pl.ds(o, 16)] for o in offs]
        area_a = [(x2[u] - x1[u]) * (y2[u] - y1[u]) for u in range(2)]
        cur = [jnp.full((16,), -1.0, jnp.float32) for _ in range(2)]
        asg = [zi, zi]
        for g in range(_G):
            gx1 = gtb_v[0, g]
            gy1 = gtb_v[1, g]
            gx2 = gtb_v[2, g]
            gy2 = gtb_v[3, g]
            ga = gtb_v[4, g]
            for u in range(2):
                w = jnp.minimum(x2[u], gx2) - jnp.maximum(x1[u], gx1)
                h = jnp.minimum(y2[u], gy2) - jnp.maximum(y1[u], gy1)
                inter = jnp.maximum(w, 0.0) * jnp.maximum(h, 0.0)
                den = area_a[u] + ga - inter + 1e-8
                iou = inter / den
                better = iou > cur[u]
                cur[u] = jnp.where(better, iou, cur[u])
                asg[u] = jnp.where(better, g, asg[u])
        for u in range(2):
            o = offs[u]
            maxov_v[pl.ds(o, 16)] = cur[u]
            assign_v[pl.ds(o, 16)] = asg[u]
            gidx = base + o + lane
            valid = gidx < _M
            fg = valid & (cur[u] > _FG_THRESH)
            bg = valid & (cur[u] < _BG_HI) & (cur[u] >= _BG_LO)
            n_fg = n_fg + jnp.where(fg, 1, 0)
            n_bg = n_bg + jnp.where(bg, 1, 0)
        return (n_fg, n_bg, n_ot)

    zero = jnp.zeros((), jnp.int32)
    af, ab, _ = lax.fori_loop(0, _NV // 2, _pass1, (zi, zi, zero))
    n_fg = jnp.sum(af)
    n_bg = jnp.sum(ab)
    nvalid = jnp.where(chunk == 7, _M - 7 * _CHUNK, _CHUNK)
    n_ot = nvalid - n_fg - n_bg

    cvec = (jnp.where(lane == 0, n_fg, 0) + jnp.where(lane == 1, n_bg, 0)
            + jnp.where(lane == 2, n_ot, 0))
    cnt_v[...] = cvec
    pltpu.sync_copy(cnt_v, counts_sp.at[pl.ds(s * 16, 16)])
    plsc.subcore_barrier()
    pltpu.sync_copy(counts_sp, cntall_v)

    rowsel = jnp.minimum(8 * im_local + lane, 15)
    in_img = lane < 8
    before = lane < chunk

    def _col(ccol):
        v = plsc.load_gather(cntall_v, [rowsel * 16 + ccol])
        vm = jnp.where(in_img, v, 0)
        tot = jnp.sum(vm)
        exc = jnp.sum(jnp.where(before, v, 0))
        evec = plsc.cumsum(vm) - vm
        return tot, exc, evec

    tot_fg, exc_fg, ev_fg = _col(0)
    tot_bg, exc_bg, ev_bg = _col(1)
    tot_ot, exc_ot, ev_ot = _col(2)

    fg_take = jnp.minimum(tot_fg, _FG_CAP)
    bg_take = jnp.minimum(tot_bg, _RB - fg_take)
    ot_take = _RB - fg_take - bg_take
    c2 = jnp.minimum(tot_ot, ot_take)
    off_bg = fg_take
    off_ot = fg_take + bg_take
    off_c3 = fg_take + bg_take + c2
    exc_c3 = (_CHUNK * chunk - jnp.minimum(fg_take, exc_fg)
              - jnp.minimum(bg_take, exc_bg) - jnp.minimum(ot_take, exc_ot))
    ev_c3 = (_CHUNK * lane - jnp.minimum(fg_take, ev_fg)
             - jnp.minimum(bg_take, ev_bg) - jnp.minimum(ot_take, ev_ot))

    exct_v[pl.ds(0, 16)] = ev_fg
    exct_v[pl.ds(16, 16)] = ev_bg
    exct_v[pl.ds(32, 16)] = ev_ot
    exct_v[pl.ds(48, 16)] = ev_c3

    def _pass2(i, carry):
        cf, cb, co, c3 = carry
        off = i * 16
        gidx = base + off + lane
        valid = gidx < _M
        cur = maxov_v[pl.ds(off, 16)]
        asg = assign_v[pl.ds(off, 16)]
        fg = valid & (cur > _FG_THRESH)
        bg = valid & (cur < _BG_HI) & (cur >= _BG_LO)
        ot = valid & jnp.logical_not(fg | bg)

        cs_fg = plsc.cumsum(jnp.where(fg, 1, 0))
        cs_bg = plsc.cumsum(jnp.where(bg, 1, 0))
        cs_ot = plsc.cumsum(jnp.where(ot, 1, 0))
        lr_fg = cf + cs_fg - 1
        lr_bg = cb + cs_bg - 1
        lr_ot = co + cs_ot - 1
        sel_fg = fg & (exc_fg + lr_fg < fg_take)
        sel_bg = bg & (exc_bg + lr_bg < bg_take)
        sel_ot = ot & (exc_ot + lr_ot < ot_take)
        c3m = valid & jnp.logical_not(sel_fg | sel_bg | sel_ot)
        cs_c3 = plsc.cumsum(jnp.where(c3m, 1, 0))
        lr_c3 = c3 + cs_c3 - 1
        sel_c3 = c3m & (off_c3 + exc_c3 + lr_c3 < _RB)
        sel = sel_fg | sel_bg | sel_ot | sel_c3

        slotrow = jnp.where(sel_fg, _SEC[0] + lr_fg,
                  jnp.where(sel_bg, _SEC[1] + lr_bg,
                  jnp.where(sel_ot, _SEC[2] + lr_ot, _SEC[3] + lr_c3)))
        saddr = slotrow * _CH

        x1 = rois_v[0, pl.ds(off, 16)]
        y1 = rois_v[1, pl.ds(off, 16)]
        x2 = rois_v[2, pl.ds(off, 16)]
        y2 = rois_v[3, pl.ds(off, 16)]
        ax1 = plsc.load_gather(gt_v, [asg])
        ay1 = plsc.load_gather(gt_v, [asg + _G])
        ax2 = plsc.load_gather(gt_v, [asg + 2 * _G])
        ay2 = plsc.load_gather(gt_v, [asg + 3 * _G])
        bird = plsc.load_gather(bird_v, [asg])
        lab = jnp.where((cur < _FG_THRESH) | ((1 + bird) == 0), 0, 1)
        labf = lab.astype(jnp.float32)

        ew = x2 - x1 + 1e-8
        eh = y2 - y1 + 1e-8
        ecx = x1 + 0.5 * ew
        ecy = y1 + 0.5 * eh
        gw = ax2 - ax1 + 1e-8
        gh = ay2 - ay1 + 1e-8
        gcx = ax1 + 0.5 * gw
        gcy = ay1 + 0.5 * gh
        dx = (gcx - ecx) / ew
        dy = (gcy - ecy) / eh
        dw = _vlog(gw / ew)
        dh = _vlog(gh / eh)
        freq = (y1 + 0.5 * (y2 - y1)) * (1.0 / 512.0)

        zf16 = jnp.zeros((16,), jnp.float32)
        for col, val in ((0, x1), (1, y1), (2, x2), (3, y2),
                         (4, zf16), (5, zf16), (6, zf16), (7, zf16),
                         (8, dx * labf), (9, dy * labf),
                         (10, dw * labf), (11, dh * labf),
                         (12, labf), (13, freq)):
            plsc.store_scatter(rowsf_v, [saddr + col], val, mask=sel)
        return (cf + cs_fg[15], cb + cs_bg[15], co + cs_ot[15], c3 + cs_c3[15])

    lax.fori_loop(0, _NV, _pass2, (zero, zero, zero, zero))

    pltpu.sync_copy(rowsf_v, big_sp.at[pl.ds(s * _BLKW, _BLKW)])
    plsc.subcore_barrier()

    copies = []
    for k2 in range(2):
        posm = s * 32 + lane + 16 * k2 - im_local * _RB
        isfg = posm < fg_take
        isbg = jnp.logical_not(isfg) & (posm < off_ot)
        isot = jnp.logical_not(isfg | isbg) & (posm < off_c3)
        rcat = posm - jnp.where(isfg, 0,
                      jnp.where(isbg, off_bg,
                      jnp.where(isot, off_ot, off_c3)))
        tbase = jnp.where(isfg, 0,
                jnp.where(isbg, 16,
                jnp.where(isot, 32, 48)))
        secb = jnp.where(isfg, _SEC[0],
               jnp.where(isbg, _SEC[1],
               jnp.where(isot, _SEC[2], _SEC[3])))
        acc = zi
        for w in range(1, 8):
            e_w = plsc.load_gather(exct_v, [tbase + w])
            acc = acc + jnp.where(e_w <= rcat, 1, 0)
        e_star = plsc.load_gather(exct_v, [tbase + acc])
        slot = rcat - e_star
        srcaddr = ((8 * im_local + acc) * _BLK + secb + slot) * _CH
        for k in range(16):
            copies.append(pltpu.async_copy(
                big_sp.at[pl.ds(pl.multiple_of(srcaddr[k], 16), 16)],
                outf_v.at[pl.ds((16 * k2 + k) * 16, 16)], sem))
    for cp in copies:
        cp.wait()

    pltpu.sync_copy(outf_v, out_hbm.at[pl.ds((c * 512 + s * 32) * _CH, 512)])


@functools.cache
def _sc_call():
    return pl.kernel(
        _emit_body,
        out_type=jax.ShapeDtypeStruct((_B * _RB * _CH,), jnp.float32),
        mesh=plsc.VectorSubcoreMesh(core_axis_name="c", subcore_axis_name="s"),
        compiler_params=pltpu.CompilerParams(needs_layout_passes=False),
        scratch_types=[
            pltpu.VMEM((4, _CHUNK), jnp.float32),
            pltpu.VMEM((4 * _G,), jnp.float32),
            pltpu.VMEM((5, _G, 16), jnp.float32),
            pltpu.VMEM((_G,), jnp.int32),
            pltpu.VMEM((_CHUNK,), jnp.float32),
            pltpu.VMEM((_CHUNK,), jnp.int32),
            pltpu.VMEM((_BLKW,), jnp.float32),
            pltpu.VMEM((64,), jnp.int32),
            pltpu.VMEM((16,), jnp.int32),
            pltpu.VMEM((256,), jnp.int32),
            pltpu.VMEM((512,), jnp.float32),
            pltpu.SemaphoreType.DMA,
            pltpu.VMEM_SHARED((256,), jnp.int32),
            pltpu.VMEM_SHARED((16 * _BLKW,), jnp.float32),
        ],
    )


def kernel(rois, gt_bbox, bird_ids, lengths):
    del lengths
    gt = gt_bbox.reshape(_B, _G, 4).astype(jnp.float32)
    allr = jnp.concatenate([rois.astype(jnp.float32), gt], axis=1)
    allr = jnp.pad(allr, ((0, 0), (0, _MP - _M), (0, 0)))
    rois_soa = allr.transpose(0, 2, 1).reshape(-1)
    gt_soa = gt.transpose(0, 2, 1).reshape(-1)
    bird = bird_ids.reshape(-1).astype(jnp.int32)
    packed = _sc_call()(rois_soa, gt_soa, bird).reshape(_B, _RB, _CH)
    out_rois = packed[:, :, 0:4]
    bbox_targets = packed[:, :, 4:12]
    out_labels = packed[:, :, 12].astype(jnp.int32)
    freq_targets = packed[:, :, 13:14]
    return (out_rois, bbox_targets, out_labels, freq_targets)
